# Initial kernel scaffold; baseline (speedup 1.0000x reference)
import jax
import jax.numpy as jnp
from jax.experimental import pallas as pl


def kernel(x, edge_index, W, attn_l, attn_r, bias, bn_gamma, bn_beta):
    raise NotImplementedError("write your pallas kernel here")


# speedup vs baseline: 37.0644x; 37.0644x over previous
import functools

import jax
import jax.numpy as jnp
from jax import lax
from jax.experimental import pallas as pl
from jax.experimental.pallas import tpu as pltpu
from jax.experimental.pallas import tpu_sc as plsc

_N = 10000
_E = 320000
_D = 128
_H = 4
_OD = 32
_HO = _H * _OD

_NC = 2
_NS = 16
_NW = _NC * _NS
_EPW = _E // _NW
_K = 80
_NCH = _EPW // _K
_G = _K // 16
_NP = 10240
_RPT = _NP // _NS
_ZR = 64
_EPS = _E // _NS
_NCH2 = _EPS // _K
_NPAIR = _NCH2 // 2
_DH = _D // 2


def _tc1_body(x_ref, w_ref, alr_ref, feat_ref, elr_ref):
    feat = jnp.dot(x_ref[...], w_ref[...], preferred_element_type=jnp.float32)
    feat_ref[...] = feat
    elr_ref[...] = jnp.dot(feat, alr_ref[...], preferred_element_type=jnp.float32)


def _sc1_body(src_hbm, dst_hbm, elr_hbm, ex_hbm, d0_hbm, d1_hbm,
              elr_v, src_v, dst_v, ex4_v, ex16_v, zb_v, den_sh):
    c = lax.axis_index("c")
    s = lax.axis_index("s")
    wid = s * _NC + c

    def _zrow(i, _):
        zb_v[i, :] = jnp.zeros((16,), jnp.float32)
        return 0
    lax.fori_loop(0, _RPT, _zrow, 0)
    pltpu.sync_copy(zb_v, den_sh.at[pl.ds(s * _RPT, _RPT)])

    def _zex(i, _):
        ex16_v[i, :] = jnp.zeros((16,), jnp.float32)
        return 0
    lax.fori_loop(0, _K, _zex, 0)

    pltpu.sync_copy(elr_hbm, elr_v)
    plsc.subcore_barrier()

    def _chunk(cc, _):
        base = wid * _EPW + cc * _K
        pltpu.sync_copy(src_hbm.at[pl.ds(base, _K)], src_v)
        pltpu.sync_copy(dst_hbm.at[pl.ds(base, _K)], dst_v)

        def _group(i, _):
            j16 = i * 16 + lax.iota(jnp.int32, 16)
            s16 = src_v[pl.ds(i * 16, 16)]
            d16 = dst_v[pl.ds(i * 16, 16)]
            for h in range(_H):
                el = plsc.load_gather(elr_v, [s16 + h * _N])
                er = plsc.load_gather(elr_v, [d16 + (4 + h) * _N])
                z = el + er
                z = jnp.maximum(z, 0.2 * z)
                exz = jnp.exp(z)
                hh = jnp.full((16,), h, jnp.int32)
                plsc.store_scatter(ex4_v, [j16, hh], exz)
                plsc.store_scatter(ex16_v, [j16, hh], exz)
            return 0
        lax.fori_loop(0, _G, _group, 0)

        pltpu.sync_copy(ex4_v, ex_hbm.at[pl.ds(base, _K)])
        pltpu.sync_copy(ex16_v, den_sh.at[dst_v], add=True)
        return 0
    lax.fori_loop(0, _NCH, _chunk, 0)

    plsc.subcore_barrier()

    @pl.when(c == 0)
    def _():
        pltpu.sync_copy(den_sh.at[pl.ds(s * _RPT, _RPT)],
                        d0_hbm.at[pl.ds(s * _RPT, _RPT)])

    @pl.when(c == 1)
    def _():
        pltpu.sync_copy(den_sh.at[pl.ds(s * _RPT, _RPT)],
                        d1_hbm.at[pl.ds(s * _RPT, _RPT)])


def _sc2_body(src_hbm, dst_hbm, ex_hbm, feat2_hbm, r0_hbm, r1_hbm,
              src0_v, src1_v, dst0_v, dst1_v, fidx0_v, fidx1_v,
              exa_v, exb_v, rows0_v, rows1_v, al_v, zb_v,
              acc_sh, sem_s, sem_b):
    c = lax.axis_index("c")
    s = lax.axis_index("s")

    src_sl = (src0_v, src1_v)
    dst_sl = (dst0_v, dst1_v)
    fidx_sl = (fidx0_v, fidx1_v)
    ex_sl = (exa_v, exb_v)
    rows_sl = (rows0_v, rows1_v)

    def _zrow(i, _):
        for q in range(4):
            zb_v[i, pl.ds(q * 16, 16)] = jnp.zeros((16,), jnp.float32)
        return 0
    lax.fori_loop(0, _ZR, _zrow, 0)
    for q in range(_RPT // _ZR):
        pltpu.sync_copy(zb_v, acc_sh.at[pl.ds(s * _RPT + q * _ZR, _ZR)])
    plsc.subcore_barrier()

    def _base(cc):
        return s * _EPS + cc * _K

    def _issue_small(cc, b):
        pltpu.async_copy(src_hbm.at[pl.ds(_base(cc), _K)], src_sl[b], sem_s)
        pltpu.async_copy(dst_hbm.at[pl.ds(_base(cc), _K)], dst_sl[b], sem_s)
        pltpu.async_copy(ex_hbm.at[pl.ds(_base(cc), _K)], ex_sl[b], sem_s)

    def _wait_small(cc, b):
        pltpu.make_async_copy(src_hbm.at[pl.ds(_base(cc), _K)], src_sl[b],
                              sem_s).wait()
        pltpu.make_async_copy(dst_hbm.at[pl.ds(_base(cc), _K)], dst_sl[b],
                              sem_s).wait()
        pltpu.make_async_copy(ex_hbm.at[pl.ds(_base(cc), _K)], ex_sl[b],
                              sem_s).wait()

    def _fidx(b):
        def _f(i, _):
            j16 = i * 16 + lax.iota(jnp.int32, 16)
            s16 = src_sl[b][pl.ds(i * 16, 16)]
            plsc.store_scatter(fidx_sl[b], [j16], s16 * 2 + c)
            return 0
        lax.fori_loop(0, _G, _f, 0)

    def _issue_big(b):
        pltpu.async_copy(feat2_hbm.at[fidx_sl[b]], rows_sl[b], sem_b)

    def _wait_big(b):
        pltpu.make_async_copy(feat2_hbm.at[fidx_sl[b]], rows_sl[b],
                              sem_b).wait()

    def _scale_scatter(b):
        def _gather_ex(i, _):
            j16 = i * 16 + lax.iota(jnp.int32, 16)
            for hh in range(2):
                hvec = jnp.zeros((16,), jnp.int32) + (2 * c + hh)
                exv = plsc.load_gather(ex_sl[b], [j16, hvec])
                plsc.store_scatter(al_v, [j16 * 2 + hh], exv)
            return 0
        lax.fori_loop(0, _G, _gather_ex, 0)

        def _edge(j, _):
            for hh in range(2):
                idx = jnp.zeros((16,), jnp.int32) + (j * 2 + hh)
                asp = plsc.load_gather(al_v, [idx])
                for q in range(2):
                    c0 = 32 * hh + 16 * q
                    rows_sl[b][j, pl.ds(c0, 16)] = (
                        rows_sl[b][j, pl.ds(c0, 16)] * asp)
            return 0
        lax.fori_loop(0, _K, _edge, 0)

        pltpu.sync_copy(rows_sl[b], acc_sh.at[dst_sl[b]], add=True)

    pltpu.sync_copy(src_hbm.at[pl.ds(_base(0), _K)], src0_v)
    pltpu.sync_copy(dst_hbm.at[pl.ds(_base(0), _K)], dst0_v)
    pltpu.sync_copy(ex_hbm.at[pl.ds(_base(0), _K)], exa_v)
    _fidx(0)
    _issue_big(0)
    _issue_small(1, 1)

    def _pair(t, _):
        c0 = 2 * t
        _wait_small(c0 + 1, 1)
        _fidx(1)
        _issue_big(1)
        _wait_big(0)
        _scale_scatter(0)

        @pl.when(t < _NPAIR - 1)
        def _():
            _issue_small(c0 + 2, 0)
            _wait_small(c0 + 2, 0)
            _fidx(0)
            _issue_big(0)

        _wait_big(1)
        _scale_scatter(1)

        @pl.when(t < _NPAIR - 1)
        def _():
            _issue_small(c0 + 3, 1)
        return 0
    lax.fori_loop(0, _NPAIR, _pair, 0)

    plsc.subcore_barrier()

    @pl.when(c == 0)
    def _():
        pltpu.sync_copy(acc_sh.at[pl.ds(s * _RPT, _RPT)],
                        r0_hbm.at[pl.ds(s * _RPT, _RPT)])

    @pl.when(c == 1)
    def _():
        pltpu.sync_copy(acc_sh.at[pl.ds(s * _RPT, _RPT)],
                        r1_hbm.at[pl.ds(s * _RPT, _RPT)])


def _tc2_body(r0_ref, r1_ref, d0_ref, d1_ref, b_ref, g_ref, be_ref, out_ref):
    den = d0_ref[...][:_N, :_H] + d1_ref[...][:_N, :_H]
    inv = 1.0 / jnp.where(den == 0.0, 1.0, den)
    halves = (r0_ref, r1_ref)
    for h in range(_H):
        cols = (h % 2) * _OD
        out_ref[:, h * _OD:(h + 1) * _OD] = (
            halves[h // 2][:_N, cols:cols + _OD] * inv[:, h:h + 1]
            + b_ref[:, h * _OD:(h + 1) * _OD])
    t = out_ref[...]
    mean = jnp.mean(t, axis=0, keepdims=True)
    d = t - mean
    var = jnp.mean(d * d, axis=0, keepdims=True)
    y = d * lax.rsqrt(var + 1e-5) * g_ref[...] + be_ref[...]
    out_ref[...] = jnp.maximum(y, 0.0)


_mesh = plsc.VectorSubcoreMesh(core_axis_name="c", subcore_axis_name="s")

_sc1 = pl.kernel(
    _sc1_body,
    out_type=(
        jax.ShapeDtypeStruct((_E, 4), jnp.float32),
        jax.ShapeDtypeStruct((_NP, 16), jnp.float32),
        jax.ShapeDtypeStruct((_NP, 16), jnp.float32),
    ),
    mesh=_mesh,
    scratch_types=[
        pltpu.VMEM((_N * 8,), jnp.float32),
        pltpu.VMEM((_K,), jnp.int32),
        pltpu.VMEM((_K,), jnp.int32),
        pltpu.VMEM((_K, 4), jnp.float32),
        pltpu.VMEM((_K, 16), jnp.float32),
        pltpu.VMEM((_RPT, 16), jnp.float32),
        pltpu.VMEM_SHARED((_NP, 16), jnp.float32),
    ],
    compiler_params=pltpu.CompilerParams(needs_layout_passes=False, use_tc_tiling_on_sc=False),
)

_sc2 = pl.kernel(
    _sc2_body,
    out_type=(
        jax.ShapeDtypeStruct((_NP, _DH), jnp.float32),
        jax.ShapeDtypeStruct((_NP, _DH), jnp.float32),
    ),
    mesh=_mesh,
    scratch_types=[
        pltpu.VMEM((_K,), jnp.int32),
        pltpu.VMEM((_K,), jnp.int32),
        pltpu.VMEM((_K,), jnp.int32),
        pltpu.VMEM((_K,), jnp.int32),
        pltpu.VMEM((_K,), jnp.int32),
        pltpu.VMEM((_K,), jnp.int32),
        pltpu.VMEM((_K, 4), jnp.float32),
        pltpu.VMEM((_K, 4), jnp.float32),
        pltpu.VMEM((_K, _DH), jnp.float32),
        pltpu.VMEM((_K, _DH), jnp.float32),
        pltpu.VMEM((_K * 2,), jnp.float32),
        pltpu.VMEM((_ZR, _DH), jnp.float32),
        pltpu.VMEM_SHARED((_NP, _DH), jnp.float32),
        pltpu.SemaphoreType.DMA,
        pltpu.SemaphoreType.DMA,
    ],
    compiler_params=pltpu.CompilerParams(needs_layout_passes=False, use_tc_tiling_on_sc=False),
)


def kernel(x, edge_index, W, attn_l, attn_r, bias, bn_gamma, bn_beta):
    src = edge_index[0]
    dst = edge_index[1]

    rows_head = jnp.arange(_D, dtype=jnp.int32) // _OD
    head_onehot = (rows_head[:, None] ==
                   jnp.arange(_H, dtype=jnp.int32)[None, :]).astype(jnp.float32)
    alr = jnp.concatenate(
        [head_onehot * attn_l.reshape(-1)[:, None],
         head_onehot * attn_r.reshape(-1)[:, None]], axis=1)

    feat, elr = pl.pallas_call(
        _tc1_body,
        out_shape=(
            jax.ShapeDtypeStruct((_N, _D), jnp.float32),
            jax.ShapeDtypeStruct((_N, 8), jnp.float32),
        ),
    )(x, W, alr)

    ex, d0, d1 = _sc1(src, dst, elr.T.reshape(-1))
    r0, r1 = _sc2(src, dst, ex, feat.reshape(2 * _N, _DH))

    out = pl.pallas_call(
        _tc2_body,
        out_shape=jax.ShapeDtypeStruct((_N, _HO), jnp.float32),
    )(r0, r1, d0, d1, bias.reshape(1, _HO), bn_gamma.reshape(1, _HO),
      bn_beta.reshape(1, _HO))
    return out
